# baseline (speedup 1.0000x reference)
import functools

import jax
import jax.numpy as jnp
from jax import lax
from jax.experimental import pallas as pl
from jax.experimental.pallas import tpu as pltpu
from jax.experimental.pallas import tpu_sc as plsc

_NUM_VOICES = 1000
_MAX_LENGTH = 510
_STYLE_DIM = 256
_BATCH = 16384

_NC, _NS = 2, 16
_NW = _NC * _NS
_BPW = _BATCH // _NW
_CHUNK = 64
_NCHUNK = _BPW // _CHUNK
_NBUF = 6
_IPC = _CHUNK // 16

_NGRP = _NUM_VOICES // 8
_SEGROWS = _MAX_LENGTH * _NUM_VOICES * 2


def _build_sc_gather():
    mesh = plsc.VectorSubcoreMesh(core_axis_name="c", subcore_axis_name="s")

    @functools.partial(
        pl.kernel,
        mesh=mesh,
        compiler_params=pltpu.CompilerParams(use_tc_tiling_on_sc=True, needs_layout_passes=False),
        out_type=jax.ShapeDtypeStruct((2 * _BATCH, 128), jnp.float32),
        scratch_types=[
            pltpu.VMEM((_BPW,), jnp.int32),
            pltpu.VMEM((_BPW,), jnp.int32),
            pltpu.VMEM((2 * _BPW,), jnp.int32),
        ]
        + [pltpu.VMEM((2 * _CHUNK, 128), jnp.float32) for _ in range(_NBUF)]
        + [pltpu.SemaphoreType.DMA for _ in range(2 * _NBUF + 2)],
    )
    def sc_gather(table_hbm, vid_hbm, len_hbm, out_hbm,
                  vid_v, len_v, idx_v, *bufs_and_sems):
        bufs = bufs_and_sems[:_NBUF]
        gsems = bufs_and_sems[_NBUF:2 * _NBUF]
        osems = bufs_and_sems[2 * _NBUF:3 * _NBUF]
        vsem, lsem = bufs_and_sems[3 * _NBUF:]

        wid = lax.axis_index("s") * _NC + lax.axis_index("c")
        base = wid * _BPW
        hv = pltpu.async_copy(vid_hbm.at[pl.ds(base, _BPW)], vid_v, vsem)
        hl = pltpu.async_copy(len_hbm.at[pl.ds(base, _BPW)], len_v, lsem)
        hv.wait()
        hl.wait()

        lanes = lax.iota(jnp.int32, 16)
        slot0 = (lanes >> 3) * 16 + (lanes & 7)

        def compute_idx(c):
            for i in range(c * _IPC, (c + 1) * _IPC):
                sl = pl.ds(i * 16, 16)
                vid = vid_v[sl]
                ln = len_v[sl]
                l = jnp.minimum(jnp.maximum(ln, 1), _MAX_LENGTH) - 1
                r0 = (l * _NGRP + (vid >> 3)) * 16 + (vid & 7)
                dst = slot0 + i * 32
                plsc.store_scatter(idx_v, [dst], r0)
                plsc.store_scatter(idx_v, [dst + 8], r0 + 8)

        def gather(c):
            return pltpu.async_copy(
                table_hbm.at[idx_v.at[pl.ds(c * 2 * _CHUNK, 2 * _CHUNK)]],
                bufs[c % _NBUF], gsems[c % _NBUF])

        def flush(c):
            return pltpu.async_copy(
                bufs[c % _NBUF],
                out_hbm.at[pl.ds(2 * base + c * 2 * _CHUNK, 2 * _CHUNK)],
                osems[c % _NBUF])

        gh = [None] * _NCHUNK
        oh = [None] * _NCHUNK
        for c in range(_NBUF):
            compute_idx(c)
            gh[c] = gather(c)
        for c in range(_NBUF, _NCHUNK):
            compute_idx(c)
        for c in range(_NCHUNK):
            gh[c].wait()
            oh[c] = flush(c)
            n = c + _NBUF - 1
            if _NBUF <= n < _NCHUNK:
                oh[n - _NBUF].wait()
                gh[n] = gather(n)
        for c in range(max(0, _NCHUNK - _NBUF), _NCHUNK):
            oh[c].wait()

    return sc_gather


_SC_GATHER = _build_sc_gather()


def kernel(voice_ids, phoneme_lengths, table):
    table_t = jnp.transpose(table, (1, 0, 2))
    table_s = table_t.reshape(_MAX_LENGTH, _NGRP, 8, 2, 128)
    table_p = jnp.transpose(table_s, (0, 1, 3, 2, 4))
    table2d = table_p.reshape(_SEGROWS, 128)
    out2 = _SC_GATHER(table2d, voice_ids, phoneme_lengths)
    out4 = out2.reshape(_BATCH // 8, 2, 8, 128)
    return jnp.transpose(out4, (0, 2, 1, 3)).reshape(_BATCH, _STYLE_DIM)


# speedup vs baseline: 1.0105x; 1.0105x over previous
import functools

import jax
import jax.numpy as jnp
from jax import lax
from jax.experimental import pallas as pl
from jax.experimental.pallas import tpu as pltpu
from jax.experimental.pallas import tpu_sc as plsc

_NUM_VOICES = 1000
_MAX_LENGTH = 510
_STYLE_DIM = 256
_BATCH = 16384

_NC, _NS = 2, 16
_NW = _NC * _NS
_BPW = _BATCH // _NW
_CHUNK = 32
_NCHUNK = _BPW // _CHUNK
_NBUF = 12
_IPC = _CHUNK // 16

_NGRP = _NUM_VOICES // 8
_SEGROWS = _MAX_LENGTH * _NUM_VOICES * 2


def _build_sc_gather():
    mesh = plsc.VectorSubcoreMesh(core_axis_name="c", subcore_axis_name="s")

    @functools.partial(
        pl.kernel,
        mesh=mesh,
        compiler_params=pltpu.CompilerParams(use_tc_tiling_on_sc=True, needs_layout_passes=False),
        out_type=jax.ShapeDtypeStruct((2 * _BATCH, 128), jnp.float32),
        scratch_types=[
            pltpu.VMEM((_BPW,), jnp.int32),
            pltpu.VMEM((_BPW,), jnp.int32),
            pltpu.VMEM((2 * _BPW,), jnp.int32),
        ]
        + [pltpu.VMEM((2 * _CHUNK, 128), jnp.float32) for _ in range(_NBUF)]
        + [pltpu.SemaphoreType.DMA for _ in range(2 * _NBUF + 2)],
    )
    def sc_gather(table_hbm, vid_hbm, len_hbm, out_hbm,
                  vid_v, len_v, idx_v, *bufs_and_sems):
        bufs = bufs_and_sems[:_NBUF]
        gsems = bufs_and_sems[_NBUF:2 * _NBUF]
        osems = bufs_and_sems[2 * _NBUF:3 * _NBUF]
        vsem, lsem = bufs_and_sems[3 * _NBUF:]

        wid = lax.axis_index("s") * _NC + lax.axis_index("c")
        base = wid * _BPW
        hv = pltpu.async_copy(vid_hbm.at[pl.ds(base, _BPW)], vid_v, vsem)
        hl = pltpu.async_copy(len_hbm.at[pl.ds(base, _BPW)], len_v, lsem)
        hv.wait()
        hl.wait()

        lanes = lax.iota(jnp.int32, 16)
        slot0 = (lanes >> 3) * 16 + (lanes & 7)

        def compute_idx(c):
            for i in range(c * _IPC, (c + 1) * _IPC):
                sl = pl.ds(i * 16, 16)
                vid = vid_v[sl]
                ln = len_v[sl]
                l = jnp.minimum(jnp.maximum(ln, 1), _MAX_LENGTH) - 1
                r0 = (l * _NGRP + (vid >> 3)) * 16 + (vid & 7)
                dst = slot0 + i * 32
                plsc.store_scatter(idx_v, [dst], r0)
                plsc.store_scatter(idx_v, [dst + 8], r0 + 8)

        def gather(c):
            return pltpu.async_copy(
                table_hbm.at[idx_v.at[pl.ds(c * 2 * _CHUNK, 2 * _CHUNK)]],
                bufs[c % _NBUF], gsems[c % _NBUF])

        def flush(c):
            return pltpu.async_copy(
                bufs[c % _NBUF],
                out_hbm.at[pl.ds(2 * base + c * 2 * _CHUNK, 2 * _CHUNK)],
                osems[c % _NBUF])

        gh = [None] * _NCHUNK
        oh = [None] * _NCHUNK
        for c in range(_NBUF):
            compute_idx(c)
            gh[c] = gather(c)
        for c in range(_NBUF, _NCHUNK):
            compute_idx(c)
        for c in range(_NCHUNK):
            gh[c].wait()
            oh[c] = flush(c)
            n = c + _NBUF - 1
            if _NBUF <= n < _NCHUNK:
                oh[n - _NBUF].wait()
                gh[n] = gather(n)
        for c in range(max(0, _NCHUNK - _NBUF), _NCHUNK):
            oh[c].wait()

    return sc_gather


_SC_GATHER = _build_sc_gather()


def kernel(voice_ids, phoneme_lengths, table):
    table_t = jnp.transpose(table, (1, 0, 2))
    table_s = table_t.reshape(_MAX_LENGTH, _NGRP, 8, 2, 128)
    table_p = jnp.transpose(table_s, (0, 1, 3, 2, 4))
    table2d = table_p.reshape(_SEGROWS, 128)
    out2 = _SC_GATHER(table2d, voice_ids, phoneme_lengths)
    out4 = out2.reshape(_BATCH // 8, 2, 8, 128)
    return jnp.transpose(out4, (0, 2, 1, 3)).reshape(_BATCH, _STYLE_DIM)


# speedup vs baseline: 1.0347x; 1.0240x over previous
import functools

import jax
import jax.numpy as jnp
from jax import lax
from jax.experimental import pallas as pl
from jax.experimental.pallas import tpu as pltpu
from jax.experimental.pallas import tpu_sc as plsc

_NUM_VOICES = 1000
_MAX_LENGTH = 510
_STYLE_DIM = 256
_BATCH = 16384

_NC, _NS = 2, 16
_NW = _NC * _NS
_BPW = _BATCH // _NW
_CHUNK = 32
_NCHUNK = _BPW // _CHUNK
_NBUF = 12
_IPC = _CHUNK // 16


def _build_sc_gather():
    mesh = plsc.VectorSubcoreMesh(core_axis_name="c", subcore_axis_name="s")

    @functools.partial(
        pl.kernel,
        mesh=mesh,
        compiler_params=pltpu.CompilerParams(use_tc_tiling_on_sc=True),
        out_type=jax.ShapeDtypeStruct((_BATCH, _STYLE_DIM), jnp.float32),
        scratch_types=[
            pltpu.VMEM((_BPW,), jnp.int32),
            pltpu.VMEM((_BPW,), jnp.int32),
            pltpu.VMEM((_BPW,), jnp.int32),
        ]
        + [pltpu.VMEM((_CHUNK, _STYLE_DIM), jnp.float32) for _ in range(_NBUF)]
        + [pltpu.SemaphoreType.DMA for _ in range(2 * _NBUF + 2)],
    )
    def sc_gather(table_hbm, vid_hbm, len_hbm, out_hbm,
                  vid_v, len_v, idx_v, *bufs_and_sems):
        bufs = bufs_and_sems[:_NBUF]
        gsems = bufs_and_sems[_NBUF:2 * _NBUF]
        osems = bufs_and_sems[2 * _NBUF:3 * _NBUF]
        vsem, lsem = bufs_and_sems[3 * _NBUF:]

        wid = lax.axis_index("s") * _NC + lax.axis_index("c")
        base = wid * _BPW
        hv = pltpu.async_copy(vid_hbm.at[pl.ds(base, _BPW)], vid_v, vsem)
        hl = pltpu.async_copy(len_hbm.at[pl.ds(base, _BPW)], len_v, lsem)
        hv.wait()
        hl.wait()

        def compute_idx(c):
            for i in range(c * _IPC, (c + 1) * _IPC):
                sl = pl.ds(i * 16, 16)
                ln = len_v[sl]
                idx = jnp.minimum(jnp.maximum(ln, 1), _MAX_LENGTH) - 1
                idx_v[sl] = idx * _NUM_VOICES + vid_v[sl]

        def gather(c):
            return pltpu.async_copy(
                table_hbm.at[idx_v.at[pl.ds(c * _CHUNK, _CHUNK)]],
                bufs[c % _NBUF], gsems[c % _NBUF])

        def flush(c):
            return pltpu.async_copy(
                bufs[c % _NBUF], out_hbm.at[pl.ds(base + c * _CHUNK, _CHUNK)],
                osems[c % _NBUF])

        gh = [None] * _NCHUNK
        oh = [None] * _NCHUNK
        for c in range(_NBUF):
            compute_idx(c)
            gh[c] = gather(c)
        for c in range(_NBUF, _NCHUNK):
            compute_idx(c)
        for c in range(_NCHUNK):
            gh[c].wait()
            oh[c] = flush(c)
            n = c + _NBUF - 1
            if _NBUF <= n < _NCHUNK:
                oh[n - _NBUF].wait()
                gh[n] = gather(n)
        for c in range(_NCHUNK - _NBUF, _NCHUNK):
            oh[c].wait()

    return sc_gather


_SC_GATHER = _build_sc_gather()


def kernel(voice_ids, phoneme_lengths, table):
    table_t = jnp.transpose(table, (1, 0, 2))
    table2d = table_t.reshape(_MAX_LENGTH * _NUM_VOICES, _STYLE_DIM)
    return _SC_GATHER(table2d, voice_ids, phoneme_lengths)
